# baseline (device time: 54450 ns/iter reference)
import jax
import jax.numpy as jnp
from jax import lax
from jax.experimental import pallas as pl
from jax.experimental.pallas import tpu as pltpu

N_DEV = 32
B, Sq, D = 2, 256, 768
Hq, Dh = 8, 64
Dq = Hq * Dh
NB = B * Hq
W_O = B * Dq
R = Sq // N_DEV
G = 4
GR = Sq // G
OPG = GR // R
SCALE = 0.125
MM = jnp.bfloat16


def _expand(a, rows):
    return jnp.concatenate(
        [jnp.broadcast_to(a[:, k : k + 1], (rows, Dh)) for k in range(NB)], axis=1
    )


def kernel(x, Wq, Wo, K_ext, V_ext):
    skv = K_ext.shape[1]

    K2 = jnp.transpose(K_ext, (0, 2, 3, 1)).reshape(NB, Dh, skv).astype(MM)
    V2 = jnp.transpose(V_ext, (0, 2, 1, 3)).reshape(NB, skv, Dh).astype(MM)

    def body(
        x_ref,
        wq_ref,
        wo_ref,
        k_ref,
        v_ref,
        out_ref,
        chunks_o,
        chunks_s,
        rs_o,
        rs_s,
        fin,
        ag_buf,
        ss1o, rs1o, ss1s, rs1s,
        ss2, rs2,
    ):
        my = lax.axis_index("i")

        barrier_sem = pltpu.get_barrier_semaphore()
        for j in range(N_DEV):
            @pl.when(my != j)
            def _():
                pl.semaphore_signal(
                    barrier_sem, inc=1,
                    device_id=(j,), device_id_type=pl.DeviceIdType.MESH,
                )

        def p1(j, send):
            dst = my if send else j
            ro = pltpu.make_async_remote_copy(
                src_ref=chunks_o.at[:, pl.ds(512 * j, 512)],
                dst_ref=rs_o.at[:, pl.ds(dst * 512, 512)],
                send_sem=ss1o.at[j],
                recv_sem=rs1o.at[dst],
                device_id=(j,),
                device_id_type=pl.DeviceIdType.MESH,
            )
            rs_ = pltpu.make_async_remote_copy(
                src_ref=chunks_s.at[:, pl.ds(128 * j, 128)],
                dst_ref=rs_s.at[:, pl.ds(dst * 128, 128)],
                send_sem=ss1s.at[j],
                recv_sem=rs1s.at[dst],
                device_id=(j,),
                device_id_type=pl.DeviceIdType.MESH,
            )
            return ro, rs_

        wq_b = wq_ref[...].astype(MM)
        for g in range(G):
            r0 = GR * g
            o_bands, m_cols, l_cols = [], [], []
            for b in range(B):
                xg = x_ref[b, r0 : r0 + GR, :].astype(MM)
                qg = jnp.dot(xg, wq_b, preferred_element_type=jnp.float32)
                for h in range(Hq):
                    bh = b * Hq + h
                    qh = qg[:, h * Dh : (h + 1) * Dh].astype(MM)
                    s = (
                        jnp.dot(qh, k_ref[bh], preferred_element_type=jnp.float32)
                        * SCALE
                    )
                    mh = jnp.max(s, axis=1, keepdims=True)
                    p = jnp.exp(s - mh)
                    lh = jnp.sum(p, axis=1, keepdims=True)
                    oh = jnp.dot(
                        p.astype(MM), v_ref[bh], preferred_element_type=jnp.float32
                    )
                    o_bands.append(oh)
                    m_cols.append(mh)
                    l_cols.append(lh)
            o_g = jnp.concatenate(o_bands, axis=1)
            m_g = jnp.concatenate(m_cols, axis=1)
            l_g = jnp.concatenate(l_cols, axis=1)
            spad = jnp.zeros((R, 128 - 2 * NB), dtype=jnp.float32)
            for jj in range(OPG):
                j = OPG * g + jj
                rsl = slice(R * jj, R * (jj + 1))
                ob = o_g[rsl]
                chunks_o[:, 512 * j : 512 * (j + 1)] = jnp.concatenate(
                    [ob[:, 0:Dq], ob[:, Dq:W_O]], axis=0
                ).astype(MM)
                chunks_s[:, 128 * j : 128 * (j + 1)] = jnp.concatenate(
                    [m_g[rsl], l_g[rsl], spad], axis=1
                )
            if g == 0:
                pl.semaphore_wait(barrier_sem, N_DEV - 1)
            for jj in range(OPG):
                j = OPG * g + jj
                @pl.when(my != j)
                def _():
                    for r in p1(j, send=True):
                        r.start()

        rs_o[:, pl.ds(my * 512, 512)] = chunks_o[:, pl.ds(my * 512, 512)]
        rs_s[:, pl.ds(my * 128, 128)] = chunks_s[:, pl.ds(my * 128, 128)]

        for s in range(N_DEV):
            @pl.when(my != s)
            def _():
                for r in p1(s, send=False):
                    r.wait_recv()

        o_parts, m_parts, l_parts = [], [], []
        for s in range(N_DEV):
            blk = rs_o[:, 512 * s : 512 * (s + 1)]
            o_parts.append(
                jnp.concatenate([blk[0:R], blk[R : 2 * R]], axis=1).astype(
                    jnp.float32
                )
            )
            sblk = rs_s[:, 128 * s : 128 * (s + 1)]
            m_parts.append(sblk[:, 0:NB])
            l_parts.append(sblk[:, NB : 2 * NB])
        o = jnp.concatenate(o_parts, axis=0)
        m = jnp.concatenate(m_parts, axis=0)
        l = jnp.concatenate(l_parts, axis=0)
        rows = Sq
        while rows > R:
            half = rows // 2
            m_new = jnp.maximum(m[:half], m[half:rows])
            a1 = jnp.exp(m[:half] - m_new)
            a2 = jnp.exp(m[half:rows] - m_new)
            l = a1 * l[:half] + a2 * l[half:rows]
            o = _expand(a1, half) * o[:half] + _expand(a2, half) * o[half:rows]
            m = m_new
            rows = half
        o_fin = o / _expand(l, R)
        fin[...] = jnp.concatenate(
            [o_fin[:, 0:Dq], o_fin[:, Dq:W_O]], axis=0
        ).astype(MM)

        def p2(j, send):
            dst = my if send else j
            return pltpu.make_async_remote_copy(
                src_ref=fin,
                dst_ref=ag_buf.at[:, pl.ds(dst * 512, 512)],
                send_sem=ss2.at[j],
                recv_sem=rs2.at[dst],
                device_id=(j,),
                device_id_type=pl.DeviceIdType.MESH,
            )

        for j in range(N_DEV):
            @pl.when(my != j)
            def _():
                p2(j, send=True).start()

        wo_b = wo_ref[...].astype(MM)

        def project(blk, row_start):
            for b in range(B):
                out_ref[b, pl.ds(row_start, R), :] = jnp.dot(
                    blk[b * R : (b + 1) * R],
                    wo_b,
                    preferred_element_type=jnp.float32,
                )

        project(fin[...], my * R)
        for s in range(N_DEV):
            @pl.when(my != s)
            def _():
                p2(s, send=False).wait_recv()
                project(ag_buf[:, 512 * s : 512 * (s + 1)], s * R)

        for j in range(N_DEV):
            @pl.when(my != j)
            def _():
                for r in p1(j, send=True):
                    r.wait_send()
                p2(j, send=True).wait_send()

    return pl.pallas_call(
        body,
        out_shape=jax.ShapeDtypeStruct((B, Sq, D), jnp.float32),
        in_specs=[pl.BlockSpec(memory_space=pltpu.VMEM)] * 5,
        out_specs=pl.BlockSpec(memory_space=pltpu.VMEM),
        scratch_shapes=[
            pltpu.VMEM((2 * R, N_DEV * Dq), MM),
            pltpu.VMEM((R, N_DEV * 128), jnp.float32),
            pltpu.VMEM((2 * R, N_DEV * Dq), MM),
            pltpu.VMEM((R, N_DEV * 128), jnp.float32),
            pltpu.VMEM((2 * R, Dq), MM),
            pltpu.VMEM((2 * R, N_DEV * Dq), MM),
            pltpu.SemaphoreType.DMA((N_DEV,)),
            pltpu.SemaphoreType.DMA((N_DEV,)),
            pltpu.SemaphoreType.DMA((N_DEV,)),
            pltpu.SemaphoreType.DMA((N_DEV,)),
            pltpu.SemaphoreType.DMA((N_DEV,)),
            pltpu.SemaphoreType.DMA((N_DEV,)),
        ],
        compiler_params=pltpu.CompilerParams(
            collective_id=0, vmem_limit_bytes=100 * 1024 * 1024
        ),
    )(x, Wq, Wo, K2, V2)


# device time: 54072 ns/iter; 1.0070x vs baseline; 1.0070x over previous
import jax
import jax.numpy as jnp
from jax import lax
from jax.experimental import pallas as pl
from jax.experimental.pallas import tpu as pltpu

N_DEV = 32
B, Sq, D = 2, 256, 768
Hq, Dh = 8, 64
Dq = Hq * Dh
NB = B * Hq
W_O = B * Dq
R = Sq // N_DEV
G = 4
GR = Sq // G
OPG = GR // R
SCALE = 0.125
MM = jnp.bfloat16


def _expand(a, rows):
    return jnp.concatenate(
        [jnp.broadcast_to(a[:, k : k + 1], (rows, Dh)) for k in range(NB)], axis=1
    )


def kernel(x, Wq, Wo, K_ext, V_ext):
    skv = K_ext.shape[1]

    K2 = jnp.transpose(K_ext, (0, 2, 3, 1)).reshape(NB, Dh, skv).astype(MM)
    V2 = jnp.transpose(V_ext, (0, 2, 1, 3)).reshape(NB, skv, Dh).astype(MM)

    def body(
        x_ref,
        wq_ref,
        wo_ref,
        k_ref,
        v_ref,
        out_ref,
        chunks_o,
        chunks_s,
        rs_o,
        rs_s,
        fin,
        ag_buf,
        ss1o, rs1o, ss1s, rs1s,
        ss2, rs2,
    ):
        my = lax.axis_index("i")

        barrier_sem = pltpu.get_barrier_semaphore()
        for j in range(N_DEV):
            @pl.when(my != j)
            def _():
                pl.semaphore_signal(
                    barrier_sem, inc=1,
                    device_id=(j,), device_id_type=pl.DeviceIdType.MESH,
                )

        def p1(j, send):
            dst = my if send else j
            ro = pltpu.make_async_remote_copy(
                src_ref=chunks_o.at[:, pl.ds(512 * j, 512)],
                dst_ref=rs_o.at[:, pl.ds(dst * 512, 512)],
                send_sem=ss1o.at[j],
                recv_sem=rs1o.at[dst],
                device_id=(j,),
                device_id_type=pl.DeviceIdType.MESH,
            )
            rs_ = pltpu.make_async_remote_copy(
                src_ref=chunks_s.at[:, pl.ds(128 * j, 128)],
                dst_ref=rs_s.at[:, pl.ds(dst * 128, 128)],
                send_sem=ss1s.at[j],
                recv_sem=rs1s.at[dst],
                device_id=(j,),
                device_id_type=pl.DeviceIdType.MESH,
            )
            return ro, rs_

        wq_b = wq_ref[...].astype(MM)
        for g in range(G):
            r0 = GR * g
            o_bands, m_cols, l_cols = [], [], []
            for b in range(B):
                xg = x_ref[b, r0 : r0 + GR, :].astype(MM)
                qg = jnp.dot(xg, wq_b, preferred_element_type=jnp.float32)
                for h in range(Hq):
                    bh = b * Hq + h
                    qh = qg[:, h * Dh : (h + 1) * Dh].astype(MM)
                    s = (
                        jnp.dot(qh, k_ref[bh], preferred_element_type=jnp.float32)
                        * SCALE
                    )
                    mh = jnp.max(s, axis=1, keepdims=True)
                    p = jnp.exp(s - mh)
                    lh = jnp.sum(p, axis=1, keepdims=True)
                    oh = jnp.dot(
                        p.astype(MM), v_ref[bh], preferred_element_type=jnp.float32
                    )
                    o_bands.append(oh)
                    m_cols.append(mh)
                    l_cols.append(lh)
            o_g = jnp.concatenate(o_bands, axis=1)
            m_g = jnp.concatenate(m_cols, axis=1)
            l_g = jnp.concatenate(l_cols, axis=1)
            spad = jnp.zeros((R, 128 - 2 * NB), dtype=jnp.float32)
            for jj in range(OPG):
                j = OPG * g + jj
                rsl = slice(R * jj, R * (jj + 1))
                ob = o_g[rsl]
                chunks_o[:, 512 * j : 512 * (j + 1)] = jnp.concatenate(
                    [ob[:, 0:Dq], ob[:, Dq:W_O]], axis=0
                ).astype(MM)
                chunks_s[:, 128 * j : 128 * (j + 1)] = jnp.concatenate(
                    [m_g[rsl], l_g[rsl], spad], axis=1
                )
            if g == 0:
                pl.semaphore_wait(barrier_sem, N_DEV - 1)
            for jj in range(OPG):
                j = OPG * g + jj
                @pl.when(my != j)
                def _():
                    for r in p1(j, send=True):
                        r.start()

        rs_o[:, pl.ds(my * 512, 512)] = chunks_o[:, pl.ds(my * 512, 512)]
        rs_s[:, pl.ds(my * 128, 128)] = chunks_s[:, pl.ds(my * 128, 128)]

        for s in range(N_DEV):
            @pl.when(my != s)
            def _():
                for r in p1(s, send=False):
                    r.wait_recv()

        o_parts, m_parts, l_parts = [], [], []
        for s in range(N_DEV):
            blk = rs_o[:, 512 * s : 512 * (s + 1)]
            o_parts.append(
                jnp.concatenate([blk[0:R], blk[R : 2 * R]], axis=1).astype(
                    jnp.float32
                )
            )
            sblk = rs_s[:, 128 * s : 128 * (s + 1)]
            m_parts.append(sblk[:, 0:NB])
            l_parts.append(sblk[:, NB : 2 * NB])
        o = jnp.concatenate(o_parts, axis=0)
        m = jnp.concatenate(m_parts, axis=0)
        l = jnp.concatenate(l_parts, axis=0)
        rows = Sq
        while rows > R:
            half = rows // 2
            m_new = jnp.maximum(m[:half], m[half:rows])
            a1 = jnp.exp(m[:half] - m_new)
            a2 = jnp.exp(m[half:rows] - m_new)
            l = a1 * l[:half] + a2 * l[half:rows]
            o = _expand(a1, half) * o[:half] + _expand(a2, half) * o[half:rows]
            m = m_new
            rows = half
        o_fin = o / _expand(l, R)
        fin[...] = jnp.concatenate(
            [o_fin[:, 0:Dq], o_fin[:, Dq:W_O]], axis=0
        ).astype(MM)

        def p2(j, send):
            dst = my if send else j
            return pltpu.make_async_remote_copy(
                src_ref=fin,
                dst_ref=ag_buf.at[:, pl.ds(dst * 512, 512)],
                send_sem=ss2.at[j],
                recv_sem=rs2.at[dst],
                device_id=(j,),
                device_id_type=pl.DeviceIdType.MESH,
            )

        for j in range(N_DEV):
            @pl.when(my != j)
            def _():
                p2(j, send=True).start()
        ag_buf[:, pl.ds(my * 512, 512)] = fin[...]

        for s in range(N_DEV):
            @pl.when(my != s)
            def _():
                p2(s, send=False).wait_recv()

        o_rows = []
        for j in range(N_DEV):
            blk = ag_buf[:, 512 * j : 512 * (j + 1)]
            o_rows.append(jnp.concatenate([blk[0:R], blk[R : 2 * R]], axis=1))
        o_full = jnp.concatenate(o_rows, axis=0)
        wo_b = wo_ref[...].astype(MM)
        for b in range(B):
            out_ref[b] = jnp.dot(
                o_full[:, b * Dq : (b + 1) * Dq],
                wo_b,
                preferred_element_type=jnp.float32,
            )

        for j in range(N_DEV):
            @pl.when(my != j)
            def _():
                for r in p1(j, send=True):
                    r.wait_send()
                p2(j, send=True).wait_send()

    return pl.pallas_call(
        body,
        out_shape=jax.ShapeDtypeStruct((B, Sq, D), jnp.float32),
        in_specs=[pl.BlockSpec(memory_space=pltpu.VMEM)] * 5,
        out_specs=pl.BlockSpec(memory_space=pltpu.VMEM),
        scratch_shapes=[
            pltpu.VMEM((2 * R, N_DEV * Dq), MM),
            pltpu.VMEM((R, N_DEV * 128), jnp.float32),
            pltpu.VMEM((2 * R, N_DEV * Dq), MM),
            pltpu.VMEM((R, N_DEV * 128), jnp.float32),
            pltpu.VMEM((2 * R, Dq), MM),
            pltpu.VMEM((2 * R, N_DEV * Dq), MM),
            pltpu.SemaphoreType.DMA((N_DEV,)),
            pltpu.SemaphoreType.DMA((N_DEV,)),
            pltpu.SemaphoreType.DMA((N_DEV,)),
            pltpu.SemaphoreType.DMA((N_DEV,)),
            pltpu.SemaphoreType.DMA((N_DEV,)),
            pltpu.SemaphoreType.DMA((N_DEV,)),
        ],
        compiler_params=pltpu.CompilerParams(
            collective_id=0, vmem_limit_bytes=100 * 1024 * 1024
        ),
    )(x, Wq, Wo, K2, V2)
